# baseline (speedup 1.0000x reference)
import functools

import jax
import jax.numpy as jnp
from jax import lax
from jax.experimental import pallas as pl
from jax.experimental.pallas import tpu as pltpu
from jax.experimental.pallas import tpu_sc as plsc

B, N, D_NODE, D_OBS, H, A = 4096, 64, 128, 128, 64, 5
BB = 64
LP = 128
NEG = -1e30


def _sc_gather_rows(adj3, agent_id):
    info = plsc.get_sparse_core_info()
    nc, ns, L = info.num_cores, info.num_subcores, info.num_lanes
    nw = nc * ns
    bpw = B // nw
    mesh = plsc.VectorSubcoreMesh(core_axis_name="c", subcore_axis_name="s")

    @functools.partial(
        pl.kernel,
        out_type=jax.ShapeDtypeStruct((B, 8, N), jnp.float32),
        mesh=mesh,
        scratch_types=[
            pltpu.VMEM((bpw,), jnp.int32),
            pltpu.SemaphoreType.DMA,
        ],
    )
    def gather_kernel(adj_hbm, ids_hbm, out_hbm, idx_v, sem):
        wid = lax.axis_index("s") * nc + lax.axis_index("c")
        base = wid * bpw
        pltpu.sync_copy(ids_hbm.at[pl.ds(base, bpw)], idx_v)

        def fire(c, carry):
            ids = idx_v[pl.ds(c * L, L)]
            for l in range(L):
                i = c * L + l
                tile = (base + i) * 8 + jnp.right_shift(ids[l], 3)
                pltpu.make_async_copy(
                    adj_hbm.at[tile], out_hbm.at[base + i], sem).start()
            return carry

        lax.fori_loop(0, bpw // L, fire, 0)
        pltpu.make_async_copy(adj_hbm.at[pl.ds(0, bpw)],
                              out_hbm.at[pl.ds(base, bpw)], sem).wait()

    return gather_kernel(adj3, agent_id)


def _tc_body(obs_ref, nobs_ref, rows8_ref, sel_ref, we_ref, be_ref, wm_ref,
             bm_ref, w1a_ref, w1b_ref, b1_ref, w2_ref, b2_ref, wact_ref,
             bact_ref, act_ref, logp_ref):
    nobs = nobs_ref[...].reshape(BB * N, D_NODE)
    h = jnp.maximum(
        jnp.dot(nobs, we_ref[...], preferred_element_type=jnp.float32)
        + be_ref[...], 0.0)
    h3 = h.reshape(BB, N, H)

    row = jnp.zeros((BB, N), jnp.float32)
    for s in range(8):
        row = row + sel_ref[:, s:s + 1] * rows8_ref[:, s, :]

    maskf = (row < 0.3).astype(jnp.float32)
    deg = jnp.maximum(jnp.sum(maskf, axis=1, keepdims=True), 1.0)
    acc = jnp.zeros((BB, H), jnp.float32)
    for j in range(N):
        acc = acc + maskf[:, j:j + 1] * h3[:, j, :]
    agg = acc / deg

    nbd = jnp.maximum(
        jnp.dot(agg, wm_ref[...], preferred_element_type=jnp.float32)
        + bm_ref[...], 0.0)
    x = jnp.maximum(
        jnp.dot(obs_ref[...], w1a_ref[...], preferred_element_type=jnp.float32)
        + jnp.dot(nbd, w1b_ref[...], preferred_element_type=jnp.float32)
        + b1_ref[...], 0.0)
    x = jnp.maximum(
        jnp.dot(x, w2_ref[...], preferred_element_type=jnp.float32)
        + b2_ref[...], 0.0)
    logits = (jnp.dot(x, wact_ref[...], preferred_element_type=jnp.float32)
              + bact_ref[...])

    col = lax.broadcasted_iota(jnp.int32, (BB, LP), 1)
    logits = jnp.where(col < A, logits, NEG)
    m = jnp.max(logits, axis=1, keepdims=True)
    idx = jnp.min(jnp.where(logits == m, col, LP), axis=1)
    ssum = jnp.sum(jnp.exp(logits - m), axis=1)
    logp = -jnp.log(ssum)

    act_ref[...] = idx.reshape(1, 1, BB)
    logp_ref[...] = logp.reshape(1, 1, BB)


def kernel(obs, node_obs, adj, agent_id, W_embed, b_embed, W_msg, b_msg,
           W1, b1, W2, b2, W_act, b_act):
    rows8 = _sc_gather_rows(adj.reshape(B * N // 8, 8, N), agent_id)
    sel = (agent_id[:, None] % 8 ==
           jnp.arange(8, dtype=jnp.int32)[None, :]).astype(jnp.float32)

    W1a, W1b = W1[:D_OBS], W1[D_OBS:]
    W_act_p = jnp.zeros((H, LP), jnp.float32).at[:, :A].set(W_act)
    b_act_p = jnp.zeros((1, LP), jnp.float32).at[0, :A].set(b_act)

    G = B // BB
    grid_spec = pl.GridSpec(
        grid=(G,),
        in_specs=[
            pl.BlockSpec((BB, D_OBS), lambda i: (i, 0)),
            pl.BlockSpec((BB, N, D_NODE), lambda i: (i, 0, 0)),
            pl.BlockSpec((BB, 8, N), lambda i: (i, 0, 0)),
            pl.BlockSpec((BB, 8), lambda i: (i, 0)),
            pl.BlockSpec((D_NODE, H), lambda i: (0, 0)),
            pl.BlockSpec((1, H), lambda i: (0, 0)),
            pl.BlockSpec((H, H), lambda i: (0, 0)),
            pl.BlockSpec((1, H), lambda i: (0, 0)),
            pl.BlockSpec((D_OBS, H), lambda i: (0, 0)),
            pl.BlockSpec((H, H), lambda i: (0, 0)),
            pl.BlockSpec((1, H), lambda i: (0, 0)),
            pl.BlockSpec((H, H), lambda i: (0, 0)),
            pl.BlockSpec((1, H), lambda i: (0, 0)),
            pl.BlockSpec((H, LP), lambda i: (0, 0)),
            pl.BlockSpec((1, LP), lambda i: (0, 0)),
        ],
        out_specs=[
            pl.BlockSpec((1, 1, BB), lambda i: (i, 0, 0)),
            pl.BlockSpec((1, 1, BB), lambda i: (i, 0, 0)),
        ],
    )
    act3, logp3 = pl.pallas_call(
        _tc_body,
        grid_spec=grid_spec,
        out_shape=[
            jax.ShapeDtypeStruct((G, 1, BB), jnp.int32),
            jax.ShapeDtypeStruct((G, 1, BB), jnp.float32),
        ],
        compiler_params=pltpu.CompilerParams(
            dimension_semantics=("arbitrary",)),
    )(obs, node_obs, rows8, sel, W_embed, b_embed.reshape(1, H), W_msg,
      b_msg.reshape(1, H), W1a, W1b, b1.reshape(1, H), W2,
      b2.reshape(1, H), W_act_p, b_act_p)

    return act3.reshape(B), logp3.reshape(B, 1)


# speedup vs baseline: 2.3349x; 2.3349x over previous
import functools

import jax
import jax.numpy as jnp
from jax import lax
from jax.experimental import pallas as pl
from jax.experimental.pallas import tpu as pltpu
from jax.experimental.pallas import tpu_sc as plsc

B, N, D_NODE, D_OBS, H, A = 4096, 64, 128, 128, 64, 5
BB = 64
LP = 128
NEG = -1e30


def _sc_gather_rows(adj2, tile_idx):
    info = plsc.get_sparse_core_info()
    nc, ns = info.num_cores, info.num_subcores
    nw = nc * ns
    bpw = B // nw
    mesh = plsc.VectorSubcoreMesh(core_axis_name="c", subcore_axis_name="s")

    @functools.partial(
        pl.kernel,
        out_type=jax.ShapeDtypeStruct((B, 8 * N), jnp.float32),
        mesh=mesh,
        scratch_types=[
            pltpu.VMEM((bpw,), jnp.int32),
            pltpu.VMEM((bpw, 8 * N), jnp.float32),
            pltpu.SemaphoreType.DMA,
        ],
    )
    def gather_kernel(adj_hbm, idx_hbm, out_hbm, idx_v, rows_v, sem):
        wid = lax.axis_index("s") * nc + lax.axis_index("c")
        base = wid * bpw
        pltpu.sync_copy(idx_hbm.at[pl.ds(base, bpw)], idx_v)
        pltpu.async_copy(adj_hbm.at[idx_v], rows_v, sem).wait()
        pltpu.sync_copy(rows_v, out_hbm.at[pl.ds(base, bpw)])

    return gather_kernel(adj2, tile_idx)


def _dot3(a, b):
    return jnp.dot(a, b, preferred_element_type=jnp.float32)


def _tc_body(obs_ref, nobs_ref, rows8_ref, sel_ref, we_ref, be_ref, wm_ref,
             bm_ref, w1a_ref, w1b_ref, b1_ref, w2_ref, b2_ref, wact_ref,
             bact_ref, act_ref, logp_ref):
    nobs = nobs_ref[...].reshape(BB * N, D_NODE)
    h = jnp.maximum(
        _dot3(nobs, we_ref[...])
        + be_ref[...], 0.0)
    h3 = h.reshape(BB, N, H)

    row = jnp.zeros((BB, N), jnp.float32)
    for s in range(8):
        row = row + sel_ref[:, s:s + 1] * rows8_ref[:, s, :]

    maskf = (row < 0.3).astype(jnp.float32)
    deg = jnp.maximum(jnp.sum(maskf, axis=1, keepdims=True), 1.0)
    acc = jnp.concatenate(
        [_dot3(maskf[b:b + 1, :], h3[b]) for b in range(BB)], axis=0)
    agg = acc / deg

    nbd = jnp.maximum(
        _dot3(agg, wm_ref[...])
        + bm_ref[...], 0.0)
    x = jnp.maximum(
        _dot3(obs_ref[...], w1a_ref[...])
        + _dot3(nbd, w1b_ref[...])
        + b1_ref[...], 0.0)
    x = jnp.maximum(
        _dot3(x, w2_ref[...])
        + b2_ref[...], 0.0)
    logits = (_dot3(x, wact_ref[...])
              + bact_ref[...])

    col = lax.broadcasted_iota(jnp.int32, (BB, LP), 1)
    logits = jnp.where(col < A, logits, NEG)
    m = jnp.max(logits, axis=1, keepdims=True)
    idx = jnp.min(jnp.where(logits == m, col, LP), axis=1)
    ssum = jnp.sum(jnp.exp(logits - m), axis=1)
    logp = -jnp.log(ssum)

    act_ref[...] = idx.reshape(1, 1, BB)
    logp_ref[...] = logp.reshape(1, 1, BB)


def kernel(obs, node_obs, adj, agent_id, W_embed, b_embed, W_msg, b_msg,
           W1, b1, W2, b2, W_act, b_act):
    tile_idx = (jnp.arange(B, dtype=jnp.int32) * 8
                + jnp.right_shift(agent_id, 3))
    rows8 = _sc_gather_rows(adj.reshape(B * N // 8, 8 * N),
                            tile_idx).reshape(B, 8, N)
    sel = (agent_id[:, None] % 8 ==
           jnp.arange(8, dtype=jnp.int32)[None, :]).astype(jnp.float32)

    W1a, W1b = W1[:D_OBS], W1[D_OBS:]
    W_act_p = jnp.zeros((H, LP), jnp.float32).at[:, :A].set(W_act)
    b_act_p = jnp.zeros((1, LP), jnp.float32).at[0, :A].set(b_act)

    G = B // BB
    grid_spec = pl.GridSpec(
        grid=(G,),
        in_specs=[
            pl.BlockSpec((BB, D_OBS), lambda i: (i, 0)),
            pl.BlockSpec((BB, N, D_NODE), lambda i: (i, 0, 0)),
            pl.BlockSpec((BB, 8, N), lambda i: (i, 0, 0)),
            pl.BlockSpec((BB, 8), lambda i: (i, 0)),
            pl.BlockSpec((D_NODE, H), lambda i: (0, 0)),
            pl.BlockSpec((1, H), lambda i: (0, 0)),
            pl.BlockSpec((H, H), lambda i: (0, 0)),
            pl.BlockSpec((1, H), lambda i: (0, 0)),
            pl.BlockSpec((D_OBS, H), lambda i: (0, 0)),
            pl.BlockSpec((H, H), lambda i: (0, 0)),
            pl.BlockSpec((1, H), lambda i: (0, 0)),
            pl.BlockSpec((H, H), lambda i: (0, 0)),
            pl.BlockSpec((1, H), lambda i: (0, 0)),
            pl.BlockSpec((H, LP), lambda i: (0, 0)),
            pl.BlockSpec((1, LP), lambda i: (0, 0)),
        ],
        out_specs=[
            pl.BlockSpec((1, 1, BB), lambda i: (i, 0, 0)),
            pl.BlockSpec((1, 1, BB), lambda i: (i, 0, 0)),
        ],
    )
    act3, logp3 = pl.pallas_call(
        _tc_body,
        grid_spec=grid_spec,
        out_shape=[
            jax.ShapeDtypeStruct((G, 1, BB), jnp.int32),
            jax.ShapeDtypeStruct((G, 1, BB), jnp.float32),
        ],
        compiler_params=pltpu.CompilerParams(
            dimension_semantics=("arbitrary",)),
    )(obs, node_obs, rows8, sel, W_embed, b_embed.reshape(1, H), W_msg,
      b_msg.reshape(1, H), W1a, W1b, b1.reshape(1, H), W2,
      b2.reshape(1, H), W_act_p, b_act_p)

    return act3.reshape(B), logp3.reshape(B, 1)
